# baseline (speedup 1.0000x reference)
import functools

import jax
import jax.numpy as jnp
from jax import lax
from jax.experimental import pallas as pl
from jax.experimental.pallas import tpu as pltpu
from jax.experimental.pallas import tpu_sc as plsc

CATE_NUM = 1000
SHOTS = 16
FEAT_DIM = 1024
FEAT_NUM = 512

CB = 40
RB = CB * SHOTS

_F16_SCALE = 1.925929944387236e-34

_SC_ROWS = 40
_SC_ACTIVE = CATE_NUM // _SC_ROWS


def _sc_scatter_res_full(res, indices):
    mesh = plsc.VectorSubcoreMesh(core_axis_name="c", subcore_axis_name="s")

    @functools.partial(
        pl.kernel,
        out_type=jax.ShapeDtypeStruct((CATE_NUM, FEAT_DIM), jnp.float32),
        mesh=mesh,
        scratch_types=[
            pltpu.VMEM((FEAT_NUM,), jnp.int32),
            pltpu.VMEM((_SC_ROWS, FEAT_NUM), jnp.float32),
            pltpu.VMEM((_SC_ROWS, FEAT_DIM), jnp.float32),
        ],
        compiler_params=pltpu.CompilerParams(use_tc_tiling_on_sc=False,
                                             needs_layout_passes=False),
    )
    def sc_kernel(res_hbm, idx_hbm, out_hbm, idx_v, rows_v, buf_v):
        w = lax.axis_index("s") * 2 + lax.axis_index("c")

        @pl.when(w < _SC_ACTIVE)
        def _():
            lo = w * _SC_ROWS
            pltpu.sync_copy(idx_hbm, idx_v)
            pltpu.sync_copy(res_hbm.at[pl.ds(lo, _SC_ROWS)], rows_v)

            zeros16 = jnp.zeros((16,), jnp.float32)

            def zbody(z, _):
                for u in range(FEAT_DIM // 16):
                    buf_v[z, pl.ds(u * 16, 16)] = zeros16
                return _

            lax.fori_loop(0, _SC_ROWS, zbody, None)

            idxs = [idx_v[pl.ds(16 * j, 16)] for j in range(FEAT_NUM // 16)]

            def row_body(t, _):
                t_vec = jnp.full((16,), t, jnp.int32)
                for j in range(FEAT_NUM // 16):
                    plsc.store_scatter(buf_v, [t_vec, idxs[j]],
                                       rows_v[t, pl.ds(16 * j, 16)])
                return _

            lax.fori_loop(0, _SC_ROWS, row_body, None)

            pltpu.sync_copy(buf_v, out_hbm.at[pl.ds(lo, _SC_ROWS)])

    return sc_kernel(res, indices)


def _bits_to_f16(b):
    h = lax.shift_right_logical(b + 0x1000, 13) & 0x7FFF
    sgn = lax.shift_right_logical(b, 16) & 0x8000
    return (sgn | h).astype(jnp.uint16)


def _to_f16(x):
    y = x * _F16_SCALE
    return _bits_to_f16(lax.bitcast_convert_type(y, jnp.int32))


def _values_body(cvt_ref, vwt_ref, out3t_ref):
    out3u = out3t_ref.bitcast(jnp.uint16)
    vw2 = vwt_ref[...] * _F16_SCALE
    for k in range(CATE_NUM // 8):
        s = slice(8 * k, 8 * k + 8)
        y = cvt_ref[s, :] * vw2
        out3u[s, :] = _bits_to_f16(lax.bitcast_convert_type(y, jnp.int32))


def _keys_body(rfull_ref, clipt_ref, ck_ref, out1_ref, out2t_ref):
    out2u = out2t_ref.bitcast(jnp.uint16)
    for k in range(CB // 8):
        s = slice(8 * k, 8 * k + 8)
        out2u[s, :] = _to_f16(clipt_ref[s, :] + rfull_ref[s, :])

    out1u = out1_ref.bitcast(jnp.uint16)
    for c in range(CB):
        s = slice(SHOTS * c, SHOTS * (c + 1))
        out1u[s, :] = _to_f16(ck_ref[s, :] + rfull_ref[c:c + 1, :])


def kernel(cache_keys, clip_weights, cache_values, res, value_weights, indices):
    res_full = _sc_scatter_res_full(res, indices)

    clipt = clip_weights.T
    cvt = cache_values.T
    vwt = value_weights.T

    grid = (CATE_NUM // CB,)
    out3t = pl.pallas_call(
        _values_body,
        grid=grid,
        in_specs=[
            pl.BlockSpec((CATE_NUM, RB), lambda i: (0, i)),
            pl.BlockSpec((1, RB), lambda i: (0, i)),
        ],
        out_specs=pl.BlockSpec((CATE_NUM, RB), lambda i: (0, i)),
        out_shape=jax.ShapeDtypeStruct((CATE_NUM, CATE_NUM * SHOTS), jnp.float16),
    )(cvt, vwt)

    out1, out2t = pl.pallas_call(
        _keys_body,
        grid=grid,
        in_specs=[
            pl.BlockSpec((CB, FEAT_DIM), lambda i: (i, 0)),
            pl.BlockSpec((CB, FEAT_DIM), lambda i: (i, 0)),
            pl.BlockSpec((RB, FEAT_DIM), lambda i: (i, 0)),
        ],
        out_specs=[
            pl.BlockSpec((RB, FEAT_DIM), lambda i: (i, 0)),
            pl.BlockSpec((CB, FEAT_DIM), lambda i: (i, 0)),
        ],
        out_shape=[
            jax.ShapeDtypeStruct((CATE_NUM * SHOTS, FEAT_DIM), jnp.float16),
            jax.ShapeDtypeStruct((CATE_NUM, FEAT_DIM), jnp.float16),
        ],
    )(res_full, clipt, cache_keys)

    return (out1, out2t.T, out3t.T)


# speedup vs baseline: 1.0545x; 1.0545x over previous
import functools

import jax
import jax.numpy as jnp
from jax import lax
from jax.experimental import pallas as pl
from jax.experimental.pallas import tpu as pltpu
from jax.experimental.pallas import tpu_sc as plsc

CATE_NUM = 1000
SHOTS = 16
FEAT_DIM = 1024
FEAT_NUM = 512

CB = 40
RB = CB * SHOTS

_F16_SCALE = 1.925929944387236e-34

_SC_ROWS = 40
_SC_ACTIVE = CATE_NUM // _SC_ROWS


def _sc_scatter_res_full(res, indices):
    mesh = plsc.VectorSubcoreMesh(core_axis_name="c", subcore_axis_name="s")

    @functools.partial(
        pl.kernel,
        out_type=jax.ShapeDtypeStruct((CATE_NUM, FEAT_DIM), jnp.float32),
        mesh=mesh,
        scratch_types=[
            pltpu.VMEM((FEAT_NUM,), jnp.int32),
            pltpu.VMEM((_SC_ROWS, FEAT_NUM), jnp.float32),
            pltpu.VMEM((_SC_ROWS, FEAT_DIM), jnp.float32),
        ],
        compiler_params=pltpu.CompilerParams(use_tc_tiling_on_sc=False,
                                             needs_layout_passes=False),
    )
    def sc_kernel(res_hbm, idx_hbm, out_hbm, idx_v, rows_v, buf_v):
        w = lax.axis_index("s") * 2 + lax.axis_index("c")

        @pl.when(w < _SC_ACTIVE)
        def _():
            lo = w * _SC_ROWS
            pltpu.sync_copy(idx_hbm, idx_v)
            pltpu.sync_copy(res_hbm.at[pl.ds(lo, _SC_ROWS)], rows_v)

            zeros16 = jnp.zeros((16,), jnp.float32)

            def zbody(z, _):
                for u in range(FEAT_DIM // 16):
                    buf_v[z, pl.ds(u * 16, 16)] = zeros16
                return _

            lax.fori_loop(0, _SC_ROWS, zbody, None)

            idxs = [idx_v[pl.ds(16 * j, 16)] for j in range(FEAT_NUM // 16)]

            def row_body(t, _):
                t_vec = jnp.full((16,), t, jnp.int32)
                for j in range(FEAT_NUM // 16):
                    plsc.store_scatter(buf_v, [t_vec, idxs[j]],
                                       rows_v[t, pl.ds(16 * j, 16)])
                return _

            lax.fori_loop(0, _SC_ROWS, row_body, None)

            pltpu.sync_copy(buf_v, out_hbm.at[pl.ds(lo, _SC_ROWS)])

    return sc_kernel(res, indices)


def _bits_to_f16(b):
    h = lax.shift_right_logical(b + 0x1000, 13) & 0x7FFF
    sgn = lax.shift_right_logical(b, 16) & 0x8000
    return (sgn | h).astype(jnp.uint16)


def _to_f16(x):
    y = x * _F16_SCALE
    return _bits_to_f16(lax.bitcast_convert_type(y, jnp.int32))


def _fused_body(rfull_ref, clipt_ref, ck_ref, cvt_ref, vwt_ref,
                out1_ref, out2t_ref, out3t_ref):
    out2u = out2t_ref.bitcast(jnp.uint16)
    for k in range(CB // 8):
        s = slice(8 * k, 8 * k + 8)
        out2u[s, :] = _to_f16(clipt_ref[s, :] + rfull_ref[s, :])

    out1u = out1_ref.bitcast(jnp.uint16)
    for c in range(CB):
        s = slice(SHOTS * c, SHOTS * (c + 1))
        out1u[s, :] = _to_f16(ck_ref[s, :] + rfull_ref[c:c + 1, :])

    out3u = out3t_ref.bitcast(jnp.uint16)
    vw2 = vwt_ref[...] * _F16_SCALE
    for k in range(CATE_NUM // 8):
        s = slice(8 * k, 8 * k + 8)
        y = cvt_ref[s, :] * vw2
        out3u[s, :] = _bits_to_f16(lax.bitcast_convert_type(y, jnp.int32))


def kernel(cache_keys, clip_weights, cache_values, res, value_weights, indices):
    res_full = _sc_scatter_res_full(res, indices)

    clipt = clip_weights.T
    cvt = cache_values.T
    vwt = value_weights.T

    grid = (CATE_NUM // CB,)
    out1, out2t, out3t = pl.pallas_call(
        _fused_body,
        grid=grid,
        in_specs=[
            pl.BlockSpec((CB, FEAT_DIM), lambda i: (i, 0)),
            pl.BlockSpec((CB, FEAT_DIM), lambda i: (i, 0)),
            pl.BlockSpec((RB, FEAT_DIM), lambda i: (i, 0)),
            pl.BlockSpec((CATE_NUM, RB), lambda i: (0, i)),
            pl.BlockSpec((1, RB), lambda i: (0, i)),
        ],
        out_specs=[
            pl.BlockSpec((RB, FEAT_DIM), lambda i: (i, 0)),
            pl.BlockSpec((CB, FEAT_DIM), lambda i: (i, 0)),
            pl.BlockSpec((CATE_NUM, RB), lambda i: (0, i)),
        ],
        out_shape=[
            jax.ShapeDtypeStruct((CATE_NUM * SHOTS, FEAT_DIM), jnp.float16),
            jax.ShapeDtypeStruct((CATE_NUM, FEAT_DIM), jnp.float16),
            jax.ShapeDtypeStruct((CATE_NUM, CATE_NUM * SHOTS), jnp.float16),
        ],
    )(res_full, clipt, cache_keys, cvt, vwt)

    return (out1, out2t.T, out3t.T)
